# Initial kernel scaffold; baseline (speedup 1.0000x reference)
import jax
import jax.numpy as jnp
from jax.experimental import pallas as pl


def kernel(h_sc, h_st, bi_e, bi_graph, sc_e, sc_graph, st_e, st_graph, params):
    raise NotImplementedError("write your pallas kernel here")


# speedup vs baseline: 2.6073x; 2.6073x over previous
import functools

import jax
import jax.numpy as jnp
from jax.experimental import pallas as pl

_B, _VSC, _VST, _H = 2, 256, 256, 128
_EPS = 1e-5
_BI = 16


def _layernorm(x, g, b):
    mu = jnp.mean(x, axis=-1, keepdims=True)
    var = jnp.mean((x - mu) ** 2, axis=-1, keepdims=True)
    return (x - mu) / jnp.sqrt(var + _EPS) * g + b


def _proj_kernel(hsc_ref, hst_ref, wsc_ref, wst_ref, bsc_ref, bst_ref,
                 osc_ref, ost_ref):
    hsc = hsc_ref[...].reshape(_B * _VSC, _H)
    hst = hst_ref[...].reshape(_B * _VST, _H)
    osc = jnp.dot(hsc, wsc_ref[...], preferred_element_type=jnp.float32)
    ost = jnp.dot(hst, wst_ref[...], preferred_element_type=jnp.float32)
    osc_ref[...] = (osc + bsc_ref[...]).reshape(_B, _VSC, 6 * _H)
    ost_ref[...] = (ost + bst_ref[...]).reshape(_B, _VST, 6 * _H)


def _edge_kernel(e_ref, arow_ref, bcol_ref, vrow_ref, wc_ref, bc_ref,
                 ge_ref, be_ref, *rest, ncols, with_col):
    if with_col:
        vcol_ref, eout_ref, row_ref, col_ref = rest
    else:
        eout_ref, row_ref = rest
    x = e_ref[0]
    xm = jnp.dot(x.reshape(_BI * ncols, _H), wc_ref[...],
                 preferred_element_type=jnp.float32)
    e_new = (xm.reshape(_BI, ncols, _H) + bc_ref[0]
             + arow_ref[0][:, None, :] + bcol_ref[0][None, :, :])
    g = jax.nn.sigmoid(e_new)
    row_ref[0] = jnp.sum(g * vrow_ref[0][None, :, :], axis=1)
    if with_col:
        part = jnp.sum(g * vcol_ref[0][:, None, :], axis=0)

        @pl.when(pl.program_id(1) == 0)
        def _():
            col_ref[0] = part

        @pl.when(pl.program_id(1) != 0)
        def _():
            col_ref[0] += part

    ln = _layernorm(e_new, ge_ref[0], be_ref[0])
    eout_ref[0] = x + jnp.maximum(ln, 0.0)


def _node_kernel(uhsc_ref, uhst_ref, st2sc_ref, sc2sc_ref, sc2st_ref,
                 st2st_ref, hsc_ref, hst_ref, gh_ref, bh_ref,
                 osc_ref, ost_ref):
    xsc = uhsc_ref[...] + st2sc_ref[...] + sc2sc_ref[...]
    xst = uhst_ref[...] + sc2st_ref[...] + st2st_ref[...]
    osc_ref[...] = hsc_ref[...] + jnp.maximum(
        _layernorm(xsc, gh_ref[0], bh_ref[0]), 0.0)
    ost_ref[...] = hst_ref[...] + jnp.maximum(
        _layernorm(xst, gh_ref[0], bh_ref[0]), 0.0)


def _edge_call(e, proj_row, proj_st_or_sc, arow_idx, bcol_idx, vrow_idx,
               wc, bc, ge, be, nrows, ncols, with_col, vcol_idx=None):
    nblk = nrows // _BI
    vec = lambda v: v.reshape(1, _H)
    small = pl.BlockSpec((1, _H), lambda b, i: (0, 0))
    full_col = lambda idx: pl.BlockSpec((1, ncols, _H),
                                        lambda b, i, idx=idx: (b, 0, idx))
    row_blk = lambda idx: pl.BlockSpec((1, _BI, _H),
                                       lambda b, i, idx=idx: (b, i, idx))
    in_specs = [
        pl.BlockSpec((1, _BI, ncols, _H), lambda b, i: (b, i, 0, 0)),
        row_blk(arow_idx),
        full_col(bcol_idx),
        full_col(vrow_idx),
        pl.BlockSpec((_H, _H), lambda b, i: (0, 0)),
        small, small, small,
    ]
    out_shapes = [
        jax.ShapeDtypeStruct((_B, nrows, ncols, _H), jnp.float32),
        jax.ShapeDtypeStruct((_B, nrows, _H), jnp.float32),
    ]
    out_specs = [
        pl.BlockSpec((1, _BI, ncols, _H), lambda b, i: (b, i, 0, 0)),
        pl.BlockSpec((1, _BI, _H), lambda b, i: (b, i, 0)),
    ]
    args = [e, proj_row, proj_st_or_sc, proj_st_or_sc, wc, vec(bc),
            vec(ge), vec(be)]
    if with_col:
        in_specs.append(row_blk(vcol_idx))
        args.append(proj_row)
        out_shapes.append(jax.ShapeDtypeStruct((_B, ncols, _H), jnp.float32))
        out_specs.append(pl.BlockSpec((1, ncols, _H), lambda b, i: (b, 0, 0)))
    return pl.pallas_call(
        functools.partial(_edge_kernel, ncols=ncols, with_col=with_col),
        grid=(_B, nblk),
        in_specs=in_specs,
        out_specs=out_specs,
        out_shape=out_shapes,
    )(*args)


def kernel(h_sc, h_st, bi_e, bi_graph, sc_e, sc_graph, st_e, st_graph,
           params):
    p = params
    wsc = jnp.concatenate([p["U1"]["w"], p["V1"]["w"], p["W1"]["w"],
                           p["bi_A"]["w"], p["sc_A"]["w"], p["sc_B"]["w"]],
                          axis=0).T
    wst = jnp.concatenate([p["U2"]["w"], p["V2"]["w"], p["W2"]["w"],
                           p["bi_B"]["w"], p["st_A"]["w"], p["st_B"]["w"]],
                          axis=0).T
    bsc = jnp.concatenate([p["U1"]["b"], p["V1"]["b"], p["W1"]["b"],
                           p["bi_A"]["b"], p["sc_A"]["b"], p["sc_B"]["b"]]
                          ).reshape(1, 6 * _H)
    bst = jnp.concatenate([p["U2"]["b"], p["V2"]["b"], p["W2"]["b"],
                           p["bi_B"]["b"], p["st_A"]["b"], p["st_B"]["b"]]
                          ).reshape(1, 6 * _H)

    proj_sc, proj_st = pl.pallas_call(
        _proj_kernel,
        out_shape=[jax.ShapeDtypeStruct((_B, _VSC, 6 * _H), jnp.float32),
                   jax.ShapeDtypeStruct((_B, _VST, 6 * _H), jnp.float32)],
    )(h_sc, h_st, wsc, wst, bsc, bst)

    ge, be = p["ln_e"]["g"], p["ln_e"]["b"]

    bi_e_out, h_st2sc, h_sc2st = _edge_call(
        bi_e, proj_sc, proj_st, arow_idx=3, bcol_idx=3, vrow_idx=1,
        wc=p["bi_C"]["w"].T, bc=p["bi_C"]["b"], ge=ge, be=be,
        nrows=_VSC, ncols=_VST, with_col=True, vcol_idx=1)
    sc_e_out, h_sc2sc = _edge_call(
        sc_e, proj_sc, proj_sc, arow_idx=4, bcol_idx=5, vrow_idx=2,
        wc=p["sc_C"]["w"].T, bc=p["sc_C"]["b"], ge=ge, be=be,
        nrows=_VSC, ncols=_VSC, with_col=False)
    st_e_out, h_st2st = _edge_call(
        st_e, proj_st, proj_st, arow_idx=4, bcol_idx=5, vrow_idx=2,
        wc=p["st_C"]["w"].T, bc=p["st_C"]["b"], ge=ge, be=be,
        nrows=_VST, ncols=_VST, with_col=False)

    full = pl.BlockSpec((_B, _VSC, _H), lambda i: (0, 0, 0))
    small = pl.BlockSpec((1, _H), lambda i: (0, 0))
    h_sc_out, h_st_out = pl.pallas_call(
        _node_kernel,
        grid=(1,),
        in_specs=[full, full, full, full, full, full, full, full,
                  small, small],
        out_specs=[full, full],
        out_shape=[jax.ShapeDtypeStruct((_B, _VSC, _H), jnp.float32),
                   jax.ShapeDtypeStruct((_B, _VST, _H), jnp.float32)],
    )(proj_sc, proj_st, h_st2sc, h_sc2sc, h_sc2st, h_st2st, h_sc, h_st,
      p["ln_h"]["g"].reshape(1, _H), p["ln_h"]["b"].reshape(1, _H))

    return (h_sc_out, h_st_out, bi_e_out, sc_e_out, st_e_out)
